# Initial kernel scaffold; baseline (speedup 1.0000x reference)
import jax
import jax.numpy as jnp
from jax.experimental import pallas as pl


def kernel(inputs, table, W, b):
    raise NotImplementedError("write your pallas kernel here")


# speedup vs baseline: 12.6814x; 12.6814x over previous
import functools

import jax
import jax.numpy as jnp
from jax import lax
from jax.experimental import pallas as pl
from jax.experimental.pallas import tpu as pltpu
from jax.experimental.pallas import tpu_sc as plsc

VOCAB = 1_000_000
EMBED = 32
BATCH = 16384
SEQ = 200
N_TOK = BATCH * SEQ

VB = 8000


def _tv_body(t_ref, w_ref, b_ref, o_ref):
    o_ref[...] = (
        jnp.dot(t_ref[...], w_ref[...], preferred_element_type=jnp.float32)
        + b_ref[0]
    )


def _compute_tv(table, W, b):
    w_scaled = W * (1.0 / SEQ)
    b_scaled = b * (1.0 / SEQ)
    return pl.pallas_call(
        _tv_body,
        grid=(VOCAB // VB,),
        in_specs=[
            pl.BlockSpec((VB, EMBED), lambda i: (i, 0)),
            pl.BlockSpec((EMBED, 1), lambda i: (0, 0)),
            pl.BlockSpec(memory_space=pltpu.SMEM),
        ],
        out_specs=pl.BlockSpec((VB, 1), lambda i: (i, 0)),
        out_shape=jax.ShapeDtypeStruct((VOCAB, 1), jnp.float32),
    )(table, w_scaled, b_scaled)


NC, NS = 2, 16
NW = NC * NS
ROWS_W = BATCH // NW
CHUNK_ROWS = 256
NCHUNK = ROWS_W // CHUNK_ROWS
CHUNK_IDX = CHUNK_ROWS * SEQ
def _make_pool():
    mesh = plsc.VectorSubcoreMesh(
        core_axis_name="c", subcore_axis_name="s", num_cores=NC, num_subcores=NS
    )
    nvec = CHUNK_ROWS // 16

    @functools.partial(
        pl.kernel,
        mesh=mesh,
        out_type=jax.ShapeDtypeStruct((BATCH,), jnp.float32),
        scratch_types=[
            pltpu.VMEM((CHUNK_IDX,), jnp.int32),
            pltpu.VMEM((CHUNK_IDX,), jnp.float32),
            pltpu.VMEM((CHUNK_ROWS,), jnp.float32),
            pltpu.SemaphoreType.DMA,
        ],
    )
    def pool(idx_hbm, tv_hbm, out_hbm, idx_v, g_v, o_v, sem):
        wid = lax.axis_index("s") * NC + lax.axis_index("c")
        for c in range(NCHUNK):
            base = wid * (NCHUNK * CHUNK_IDX) + c * CHUNK_IDX
            pltpu.sync_copy(idx_hbm.at[pl.ds(base, CHUNK_IDX)], idx_v)
            pltpu.async_copy(tv_hbm.at[idx_v], g_v, sem).wait()

            def body(t, accs):
                off = t * CHUNK_ROWS
                return tuple(
                    accs[i] + g_v[pl.ds(off + 16 * i, 16)] for i in range(nvec)
                )

            accs = lax.fori_loop(
                0, SEQ, body, (jnp.zeros((16,), jnp.float32),) * nvec
            )
            for i in range(nvec):
                o_v[pl.ds(16 * i, 16)] = 1.0 / (1.0 + jnp.exp(-accs[i]))
            pltpu.sync_copy(
                o_v, out_hbm.at[pl.ds(wid * ROWS_W + c * CHUNK_ROWS, CHUNK_ROWS)]
            )

    return pool


_pool_kernel = _make_pool()


def kernel(inputs, table, W, b):
    tv = _compute_tv(table, W, b)
    idx = (
        inputs.astype(jnp.int32)
        .reshape(BATCH // CHUNK_ROWS, CHUNK_ROWS, SEQ)
        .transpose(0, 2, 1)
        .reshape(N_TOK)
    )
    out = _pool_kernel(idx, tv.reshape(VOCAB))
    return out.reshape(BATCH, 1)


# speedup vs baseline: 13.9940x; 1.1035x over previous
import functools

import jax
import jax.numpy as jnp
from jax import lax
from jax.experimental import pallas as pl
from jax.experimental.pallas import tpu as pltpu
from jax.experimental.pallas import tpu_sc as plsc

VOCAB = 1_000_000
EMBED = 32
BATCH = 16384
SEQ = 200
N_TOK = BATCH * SEQ

GRP = 64
TV_ROWS = VOCAB // GRP
RB = 128


def _tv_body(t_ref, w_ref, b_ref, o_ref):
    o_ref[...] = (
        jnp.dot(t_ref[...], w_ref[...], preferred_element_type=jnp.float32)
        + b_ref[0]
    )


def _compute_tv(table, W, b):
    w_scaled = W.reshape(EMBED) * (1.0 / SEQ)
    w_kron = jnp.kron(jnp.eye(GRP, dtype=jnp.float32), w_scaled[:, None])
    b_scaled = b * (1.0 / SEQ)
    tv2d = pl.pallas_call(
        _tv_body,
        grid=((TV_ROWS + RB - 1) // RB,),
        in_specs=[
            pl.BlockSpec((RB, GRP * EMBED), lambda i: (i, 0)),
            pl.BlockSpec((GRP * EMBED, GRP), lambda i: (0, 0)),
            pl.BlockSpec(memory_space=pltpu.SMEM),
        ],
        out_specs=pl.BlockSpec((RB, GRP), lambda i: (i, 0)),
        out_shape=jax.ShapeDtypeStruct((TV_ROWS, GRP), jnp.float32),
    )(table.reshape(TV_ROWS, GRP * EMBED), w_kron, b_scaled)
    return tv2d.reshape(VOCAB)


NC, NS = 2, 16
NW = NC * NS
ROWS_W = BATCH // NW
CHUNK_ROWS = 256
NCHUNK = ROWS_W // CHUNK_ROWS
CHUNK_IDX = CHUNK_ROWS * SEQ
NACC = CHUNK_ROWS // 16


def _make_pool():
    mesh = plsc.VectorSubcoreMesh(
        core_axis_name="c", subcore_axis_name="s", num_cores=NC, num_subcores=NS
    )
    @functools.partial(
        pl.kernel,
        mesh=mesh,
        out_type=jax.ShapeDtypeStruct((BATCH,), jnp.float32),
        scratch_types=[
            pltpu.VMEM((CHUNK_IDX,), jnp.int32),
            pltpu.VMEM((CHUNK_IDX,), jnp.float32),
            pltpu.VMEM((CHUNK_ROWS,), jnp.float32),
            pltpu.SemaphoreType.DMA,
        ],
    )
    def pool(idx_hbm, tv_hbm, out_hbm, idx_v, g_v, o_v, sem):
        wid = lax.axis_index("s") * NC + lax.axis_index("c")
        for c in range(NCHUNK):
            base = wid * (NCHUNK * CHUNK_IDX) + c * CHUNK_IDX
            pltpu.sync_copy(idx_hbm.at[pl.ds(base, CHUNK_IDX)], idx_v)
            pltpu.async_copy(tv_hbm.at[idx_v], g_v, sem).wait()

            for j in range(NACC):
                def tok_add(t, acc, j=j):
                    return acc + g_v[pl.ds(t * CHUNK_ROWS + j * 16, 16)]
                acc = lax.fori_loop(
                    0, SEQ, tok_add, jnp.zeros((16,), jnp.float32)
                )
                o_v[pl.ds(j * 16, 16)] = 1.0 / (1.0 + jnp.exp(-acc))

            pltpu.sync_copy(
                o_v, out_hbm.at[pl.ds(wid * ROWS_W + c * CHUNK_ROWS, CHUNK_ROWS)]
            )

    return pool


_pool_kernel = _make_pool()


def kernel(inputs, table, W, b):
    tv = _compute_tv(table, W, b)
    idx = (
        inputs.astype(jnp.int32)
        .reshape(NW, NCHUNK, CHUNK_ROWS, SEQ)
        .transpose(0, 1, 3, 2)
        .reshape(N_TOK)
    )
    out = _pool_kernel(idx, tv)
    return out.reshape(BATCH, 1)


# speedup vs baseline: 14.4967x; 1.0359x over previous
import functools

import jax
import jax.numpy as jnp
from jax import lax
from jax.experimental import pallas as pl
from jax.experimental.pallas import tpu as pltpu
from jax.experimental.pallas import tpu_sc as plsc

VOCAB = 1_000_000
EMBED = 32
BATCH = 16384
SEQ = 200
N_TOK = BATCH * SEQ

GRP = 64
TV_ROWS = VOCAB // GRP
RB = 128


def _tv_body(t_ref, w_ref, b_ref, o_ref):
    o_ref[...] = (
        jnp.dot(t_ref[...], w_ref[...], preferred_element_type=jnp.float32)
        + b_ref[0]
    )


def _compute_tv(table, W, b):
    w_scaled = W.reshape(EMBED) * (1.0 / SEQ)
    w_kron = jnp.kron(jnp.eye(GRP, dtype=jnp.float32), w_scaled[:, None])
    b_scaled = b * (1.0 / SEQ)
    tv2d = pl.pallas_call(
        _tv_body,
        grid=((TV_ROWS + RB - 1) // RB,),
        in_specs=[
            pl.BlockSpec((RB, GRP * EMBED), lambda i: (i, 0)),
            pl.BlockSpec((GRP * EMBED, GRP), lambda i: (0, 0)),
            pl.BlockSpec(memory_space=pltpu.SMEM),
        ],
        out_specs=pl.BlockSpec((RB, GRP), lambda i: (i, 0)),
        out_shape=jax.ShapeDtypeStruct((TV_ROWS, GRP), jnp.float32),
    )(table.reshape(TV_ROWS, GRP * EMBED), w_kron, b_scaled)
    return tv2d.reshape(VOCAB)


NGRP_T = BATCH // 256


def _tr_body(i_ref, o_ref):
    o_ref[...] = i_ref[...].T


def _permute_idx(idx2d):
    out = pl.pallas_call(
        _tr_body,
        grid=(NGRP_T,),
        in_specs=[pl.BlockSpec((256, SEQ), lambda i: (i, 0))],
        out_specs=pl.BlockSpec((SEQ, 256), lambda i: (i, 0)),
        out_shape=jax.ShapeDtypeStruct((NGRP_T * SEQ, 256), jnp.int32),
    )(idx2d)
    return out.reshape(N_TOK)


NC, NS = 2, 16
NW = NC * NS
ROWS_W = BATCH // NW
CHUNK_ROWS = 256
NCHUNK = ROWS_W // CHUNK_ROWS
CHUNK_IDX = CHUNK_ROWS * SEQ
NACC = CHUNK_ROWS // 16


def _make_pool():
    mesh = plsc.VectorSubcoreMesh(
        core_axis_name="c", subcore_axis_name="s", num_cores=NC, num_subcores=NS
    )
    @functools.partial(
        pl.kernel,
        mesh=mesh,
        out_type=jax.ShapeDtypeStruct((BATCH,), jnp.float32),
        scratch_types=[
            pltpu.VMEM((CHUNK_IDX,), jnp.int32),
            pltpu.VMEM((CHUNK_IDX,), jnp.float32),
            pltpu.VMEM((CHUNK_ROWS,), jnp.float32),
            pltpu.SemaphoreType.DMA,
        ],
    )
    def pool(idx_hbm, tv_hbm, out_hbm, idx_v, g_v, o_v, sem):
        wid = lax.axis_index("s") * NC + lax.axis_index("c")
        for c in range(NCHUNK):
            base = wid * (NCHUNK * CHUNK_IDX) + c * CHUNK_IDX
            pltpu.sync_copy(idx_hbm.at[pl.ds(base, CHUNK_IDX)], idx_v)
            pltpu.async_copy(tv_hbm.at[idx_v], g_v, sem).wait()

            for j in range(NACC):
                def tok_add(t, acc, j=j):
                    return acc + g_v[pl.ds(t * CHUNK_ROWS + j * 16, 16)]
                acc = lax.fori_loop(
                    0, SEQ, tok_add, jnp.zeros((16,), jnp.float32)
                )
                o_v[pl.ds(j * 16, 16)] = 1.0 / (1.0 + jnp.exp(-acc))

            pltpu.sync_copy(
                o_v, out_hbm.at[pl.ds(wid * ROWS_W + c * CHUNK_ROWS, CHUNK_ROWS)]
            )

    return pool


_pool_kernel = _make_pool()


def kernel(inputs, table, W, b):
    tv = _compute_tv(table, W, b)
    idx = _permute_idx(inputs.astype(jnp.int32))
    out = _pool_kernel(idx, tv)
    return out.reshape(BATCH, 1)


# speedup vs baseline: 14.5180x; 1.0015x over previous
import functools

import jax
import jax.numpy as jnp
from jax import lax
from jax.experimental import pallas as pl
from jax.experimental.pallas import tpu as pltpu
from jax.experimental.pallas import tpu_sc as plsc

VOCAB = 1_000_000
EMBED = 32
BATCH = 16384
SEQ = 200
N_TOK = BATCH * SEQ

GRP = 64
TV_ROWS = VOCAB // GRP
RB = 128


def _tv_body(t_ref, w_ref, b_ref, o_ref):
    o_ref[...] = (
        jnp.dot(t_ref[...], w_ref[...], preferred_element_type=jnp.float32)
        + b_ref[0]
    )


def _compute_tv(table, W, b):
    w_scaled = W.reshape(EMBED) * (1.0 / SEQ)
    w_kron = jnp.kron(jnp.eye(GRP, dtype=jnp.float32), w_scaled[:, None])
    b_scaled = b * (1.0 / SEQ)
    tv2d = pl.pallas_call(
        _tv_body,
        grid=((TV_ROWS + RB - 1) // RB,),
        in_specs=[
            pl.BlockSpec((RB, GRP * EMBED), lambda i: (i, 0)),
            pl.BlockSpec((GRP * EMBED, GRP), lambda i: (0, 0)),
            pl.BlockSpec(memory_space=pltpu.SMEM),
        ],
        out_specs=pl.BlockSpec((RB, GRP), lambda i: (i, 0)),
        out_shape=jax.ShapeDtypeStruct((TV_ROWS, GRP), jnp.float32),
    )(table.reshape(TV_ROWS, GRP * EMBED), w_kron, b_scaled)
    return tv2d.reshape(VOCAB)


NGRP_T = BATCH // 128


def _tr_body(i_ref, o_ref):
    o_ref[...] = i_ref[...].T


def _permute_idx(idx2d):
    out = pl.pallas_call(
        _tr_body,
        grid=(NGRP_T,),
        in_specs=[pl.BlockSpec((128, SEQ), lambda i: (i, 0))],
        out_specs=pl.BlockSpec((SEQ, 128), lambda i: (i, 0)),
        out_shape=jax.ShapeDtypeStruct((NGRP_T * SEQ, 128), jnp.int32),
    )(idx2d)
    return out.reshape(N_TOK)


NC, NS = 2, 16
NW = NC * NS
ROWS_W = BATCH // NW
CHUNK_ROWS = 128
NCHUNK = ROWS_W // CHUNK_ROWS
CHUNK_IDX = CHUNK_ROWS * SEQ
NACC = CHUNK_ROWS // 16


def _make_pool():
    mesh = plsc.VectorSubcoreMesh(
        core_axis_name="c", subcore_axis_name="s", num_cores=NC, num_subcores=NS
    )
    @functools.partial(
        pl.kernel,
        mesh=mesh,
        out_type=jax.ShapeDtypeStruct((BATCH,), jnp.float32),
        scratch_types=[
            pltpu.VMEM((CHUNK_IDX,), jnp.int32),
            pltpu.VMEM((CHUNK_IDX,), jnp.float32),
            pltpu.VMEM((CHUNK_ROWS,), jnp.float32),
            pltpu.SemaphoreType.DMA,
        ],
    )
    def pool(idx_hbm, tv_hbm, out_hbm, idx_v, g_v, o_v, sem):
        wid = lax.axis_index("s") * NC + lax.axis_index("c")
        for c in range(NCHUNK):
            base = wid * (NCHUNK * CHUNK_IDX) + c * CHUNK_IDX
            pltpu.sync_copy(idx_hbm.at[pl.ds(base, CHUNK_IDX)], idx_v)
            pltpu.async_copy(tv_hbm.at[idx_v], g_v, sem).wait()

            for j in range(NACC):
                def tok_add(t, acc, j=j):
                    return acc + g_v[pl.ds(t * CHUNK_ROWS + j * 16, 16)]
                acc = lax.fori_loop(
                    0, SEQ, tok_add, jnp.zeros((16,), jnp.float32)
                )
                o_v[pl.ds(j * 16, 16)] = 1.0 / (1.0 + jnp.exp(-acc))

            pltpu.sync_copy(
                o_v, out_hbm.at[pl.ds(wid * ROWS_W + c * CHUNK_ROWS, CHUNK_ROWS)]
            )

    return pool


_pool_kernel = _make_pool()


def kernel(inputs, table, W, b):
    tv = _compute_tv(table, W, b)
    idx = _permute_idx(inputs.astype(jnp.int32))
    out = _pool_kernel(idx, tv)
    return out.reshape(BATCH, 1)


# speedup vs baseline: 16.6335x; 1.1457x over previous
import functools

import jax
import jax.numpy as jnp
from jax import lax
from jax.experimental import pallas as pl
from jax.experimental.pallas import tpu as pltpu
from jax.experimental.pallas import tpu_sc as plsc

VOCAB = 1_000_000
EMBED = 32
BATCH = 16384
SEQ = 200
N_TOK = BATCH * SEQ

TBLK = 8192


def _tv_body(t_ref, w_ref, b_ref, o_ref):
    y = jnp.dot(t_ref[...], w_ref[...], preferred_element_type=jnp.float32)
    o_ref[...] = (y.T + b_ref[0])[0]


def _compute_tv(table, W, b):
    w_scaled = W.reshape(EMBED, 1) * (1.0 / SEQ)
    b_scaled = b * (1.0 / SEQ)
    return pl.pallas_call(
        _tv_body,
        grid=((VOCAB + TBLK - 1) // TBLK,),
        in_specs=[
            pl.BlockSpec((TBLK, EMBED), lambda i: (i, 0)),
            pl.BlockSpec((EMBED, 1), lambda i: (0, 0)),
            pl.BlockSpec(memory_space=pltpu.SMEM),
        ],
        out_specs=pl.BlockSpec((TBLK,), lambda i: (i,)),
        out_shape=jax.ShapeDtypeStruct((VOCAB,), jnp.float32),
    )(table, w_scaled, b_scaled)


NGRP_T = BATCH // 128
TR_GRPS = 8


def _tr_body(i_ref, o_ref):
    for g in range(TR_GRPS):
        o_ref[g * SEQ:(g + 1) * SEQ, :] = i_ref[g * 128:(g + 1) * 128, :].T


def _permute_idx(idx2d):
    out = pl.pallas_call(
        _tr_body,
        grid=(NGRP_T // TR_GRPS,),
        in_specs=[pl.BlockSpec((TR_GRPS * 128, SEQ), lambda i: (i, 0))],
        out_specs=pl.BlockSpec((TR_GRPS * SEQ, 128), lambda i: (i, 0)),
        out_shape=jax.ShapeDtypeStruct((NGRP_T * SEQ, 128), jnp.int32),
    )(idx2d)
    return out.reshape(N_TOK)


NC, NS = 2, 16
NW = NC * NS
ROWS_W = BATCH // NW
CHUNK_ROWS = 128
NCHUNK = ROWS_W // CHUNK_ROWS
CHUNK_IDX = CHUNK_ROWS * SEQ
NACC = CHUNK_ROWS // 16


def _make_pool():
    mesh = plsc.VectorSubcoreMesh(
        core_axis_name="c", subcore_axis_name="s", num_cores=NC, num_subcores=NS
    )
    @functools.partial(
        pl.kernel,
        mesh=mesh,
        out_type=jax.ShapeDtypeStruct((BATCH,), jnp.float32),
        scratch_types=[
            pltpu.VMEM((CHUNK_IDX,), jnp.int32),
            pltpu.VMEM((CHUNK_IDX,), jnp.float32),
            pltpu.VMEM((CHUNK_ROWS,), jnp.float32),
            pltpu.SemaphoreType.DMA,
        ],
    )
    def pool(idx_hbm, tv_hbm, out_hbm, idx_v, g_v, o_v, sem):
        wid = lax.axis_index("s") * NC + lax.axis_index("c")
        for c in range(NCHUNK):
            base = wid * (NCHUNK * CHUNK_IDX) + c * CHUNK_IDX
            pltpu.sync_copy(idx_hbm.at[pl.ds(base, CHUNK_IDX)], idx_v)
            pltpu.async_copy(tv_hbm.at[idx_v], g_v, sem).wait()

            for j in range(NACC):
                def tok_add(t, acc, j=j):
                    return acc + g_v[pl.ds(t * CHUNK_ROWS + j * 16, 16)]
                acc = lax.fori_loop(
                    0, SEQ, tok_add, jnp.zeros((16,), jnp.float32)
                )
                o_v[pl.ds(j * 16, 16)] = 1.0 / (1.0 + jnp.exp(-acc))

            pltpu.sync_copy(
                o_v, out_hbm.at[pl.ds(wid * ROWS_W + c * CHUNK_ROWS, CHUNK_ROWS)]
            )

    return pool


_pool_kernel = _make_pool()


def kernel(inputs, table, W, b):
    tv = _compute_tv(table, W, b)
    idx = _permute_idx(inputs.astype(jnp.int32))
    out = _pool_kernel(idx, tv)
    return out.reshape(BATCH, 1)
